# baseline (speedup 1.0000x reference)
import jax
import jax.numpy as jnp
from jax.experimental import pallas as pl

_NUM_CLASSES = 1000
_MULT = 26
_ROWS = 1024
_BLOCK_ROWS = 128


def _onehot_block(feat_ref, out_ref):
    f = feat_ref[...]
    classes = jax.lax.broadcasted_iota(
        jnp.int32, (_BLOCK_ROWS, _MULT, _NUM_CLASSES), 2
    )
    out_ref[...] = (f[:, :, None] == classes).astype(jnp.int32)


def kernel(feat):
    grid = (_ROWS // _BLOCK_ROWS,)
    return pl.pallas_call(
        _onehot_block,
        grid=grid,
        in_specs=[pl.BlockSpec((_BLOCK_ROWS, _MULT), lambda i: (i, 0))],
        out_specs=pl.BlockSpec(
            (_BLOCK_ROWS, _MULT, _NUM_CLASSES), lambda i: (i, 0, 0)
        ),
        out_shape=jax.ShapeDtypeStruct((_ROWS, _MULT, _NUM_CLASSES), jnp.int32),
    )(feat)
